# baseline (speedup 1.0000x reference)
import functools

import jax
import jax.numpy as jnp
from jax.experimental import pallas as pl
from jax.experimental.pallas import tpu as pltpu


def _ln_relu_body(x_ref, w_ref, b_ref, o_ref, *, eps, inv_c):
    blk, r, c = x_ref.shape
    x = x_ref[...].reshape(blk * r, c)
    center = jnp.eye(c, dtype=jnp.float32) - inv_c
    ones_half = jnp.full((c, 128), inv_c, jnp.float32)
    xc = jax.lax.dot_general(x, center, (((1,), (0,)), ((), ())),
                             preferred_element_type=jnp.float32)
    v128 = jax.lax.dot_general(xc * xc, ones_half, (((1,), (0,)), ((), ())),
                               preferred_element_type=jnp.float32)
    inv128 = jax.lax.rsqrt(v128 + eps)
    inv = jnp.concatenate([inv128, inv128], axis=1)
    w = w_ref[...]
    b = b_ref[...]
    y = xc * inv * w + b
    o_ref[...] = jnp.maximum(y, 0.0).reshape(blk, r, c)


def kernel(x, weight, bias):
    n, c, h, w = x.shape
    hw = h * w
    xt = jnp.transpose(x, (0, 2, 3, 1)).reshape(n, hw, c)
    wc = weight.reshape(1, c).astype(jnp.float32)
    bc = bias.reshape(1, c).astype(jnp.float32)
    out = pl.pallas_call(
        functools.partial(_ln_relu_body, eps=1e-5, inv_c=1.0 / c),
        out_shape=jax.ShapeDtypeStruct((n, hw, c), x.dtype),
        grid=(n // 2,),
        in_specs=[
            pl.BlockSpec((2, hw, c), lambda i: (i, 0, 0)),
            pl.BlockSpec((1, c), lambda i: (0, 0)),
            pl.BlockSpec((1, c), lambda i: (0, 0)),
        ],
        out_specs=pl.BlockSpec((2, hw, c), lambda i: (i, 0, 0)),
        compiler_params=pltpu.CompilerParams(
            dimension_semantics=("parallel",),
            vmem_limit_bytes=100 * 1024 * 1024,
        ),
    )(xt, wc, bc)
    return jnp.transpose(out.reshape(n, h, w, c), (0, 3, 1, 2))


# speedup vs baseline: 1.0036x; 1.0036x over previous
import functools

import jax
import jax.numpy as jnp
from jax.experimental import pallas as pl
from jax.experimental.pallas import tpu as pltpu


def _ln_relu_body(x_ref, w_ref, b_ref, o_ref, *, eps, inv_c):
    blk, r, c = x_ref.shape
    x = x_ref[...].reshape(blk * r, c)
    ones_full = jnp.full((c, c), inv_c, jnp.float32)
    ones_half = jnp.full((c, 128), inv_c, jnp.float32)
    mean = jax.lax.dot_general(x, ones_full, (((1,), (0,)), ((), ())),
                               preferred_element_type=jnp.float32)
    ex2 = jax.lax.dot_general(x * x, ones_half, (((1,), (0,)), ((), ())),
                              preferred_element_type=jnp.float32)
    m128 = mean[:, :128]
    inv128 = jax.lax.rsqrt(ex2 - m128 * m128 + eps)
    inv = jnp.concatenate([inv128, inv128], axis=1)
    w = w_ref[...]
    b = b_ref[...]
    y = (x - mean) * inv * w + b
    o_ref[...] = jnp.maximum(y, 0.0).reshape(blk, r, c)


def kernel(x, weight, bias):
    n, c, h, w = x.shape
    hw = h * w
    xt = jnp.transpose(x, (0, 2, 3, 1)).reshape(n, hw, c)
    wc = weight.reshape(1, c).astype(jnp.float32)
    bc = bias.reshape(1, c).astype(jnp.float32)
    out = pl.pallas_call(
        functools.partial(_ln_relu_body, eps=1e-5, inv_c=1.0 / c),
        out_shape=jax.ShapeDtypeStruct((n, hw, c), x.dtype),
        grid=(n // 2,),
        in_specs=[
            pl.BlockSpec((2, hw, c), lambda i: (i, 0, 0)),
            pl.BlockSpec((1, c), lambda i: (0, 0)),
            pl.BlockSpec((1, c), lambda i: (0, 0)),
        ],
        out_specs=pl.BlockSpec((2, hw, c), lambda i: (i, 0, 0)),
        compiler_params=pltpu.CompilerParams(
            dimension_semantics=("parallel",),
            vmem_limit_bytes=100 * 1024 * 1024,
        ),
    )(xt, wc, bc)
    return jnp.transpose(out.reshape(n, h, w, c), (0, 3, 1, 2))
